# baseline (speedup 1.0000x reference)
import functools

import jax
import jax.numpy as jnp
from jax import lax
from jax.experimental import pallas as pl
from jax.experimental.pallas import tpu as pltpu
from jax.experimental.pallas import tpu_sc as plsc

N = 10000
E = 160000
D_IN = 256
D_OUT = 256
H = 128
NT = 16
NC = 2
CH = 64
NCHUNK = 162
G = 6
NG = NCHUNK // G
GE = G * CH
EPT = CH * NCHUNK
EPAD = NT * EPT
ROWS_PT = 624
BR = 640
TAIL0 = NT * ROWS_PT
TAIL = N - TAIL0
MM_BLK = 1000
MM_NB = N // MM_BLK


def _mm_body(x_ref, w_ref, o_ref):
    o_ref[...] = jnp.dot(x_ref[...], w_ref[...],
                         preferred_element_type=jnp.float32)


def _matmul_halves(x, W):
    return pl.pallas_call(
        _mm_body,
        grid=(MM_NB, NC),
        in_specs=[
            pl.BlockSpec((MM_BLK, D_IN), lambda i, h: (i, 0)),
            pl.BlockSpec((D_IN, H), lambda i, h: (0, h)),
        ],
        out_specs=pl.BlockSpec((MM_BLK, H), lambda i, h: (h * MM_NB + i, 0)),
        out_shape=jax.ShapeDtypeStruct((NC * N, H), jnp.float32),
    )(x, W)


_MESH = plsc.VectorSubcoreMesh(core_axis_name="c", subcore_axis_name="s")


@functools.partial(
    pl.kernel,
    out_type=jax.ShapeDtypeStruct((N, D_OUT), jnp.float32),
    mesh=_MESH,
    compiler_params=pltpu.CompilerParams(internal_scratch_in_bytes=4096),
    scratch_types=[
        pltpu.VMEM((3, CH, H), jnp.float32),
        pltpu.VMEM((2, CH, H), jnp.float32),
        pltpu.VMEM((2, G, CH), jnp.int32),
        pltpu.VMEM((2, G, CH), jnp.int32),
        pltpu.VMEM((2 * GE,), jnp.float32),
        pltpu.VMEM_SHARED((N, H), jnp.float32),
        pltpu.SemaphoreType.DMA,
        pltpu.SemaphoreType.DMA,
        pltpu.SemaphoreType.DMA,
        pltpu.SemaphoreType.DMA,
        pltpu.SemaphoreType.DMA,
    ],
)
def _sc_aggregate(xw_hbm, srcb_hbm, dst_hbm, w_hbm, brep_hbm, out_hbm,
                  gbuf, sbuf, srcg, dstg, wgf, acc,
                  gsem0, gsem1, gsem2, ssem0, ssem1):
    c = lax.axis_index("c")
    s = lax.axis_index("s")
    gsems = (gsem0, gsem1, gsem2)
    ssems = (ssem0, ssem1)

    row0 = s * ROWS_PT
    pltpu.sync_copy(brep_hbm.at[c, pl.ds(0, ROWS_PT)],
                    acc.at[pl.ds(row0, ROWS_PT)])

    @pl.when(s == NT - 1)
    def _init_tail():
        pltpu.sync_copy(brep_hbm.at[c, pl.ds(ROWS_PT, TAIL)],
                        acc.at[pl.ds(TAIL0, TAIL)])

    pltpu.sync_copy(srcb_hbm.at[c, s, 0], srcg.at[0])
    pltpu.sync_copy(dst_hbm.at[s, 0], dstg.at[0])
    pltpu.sync_copy(w_hbm.at[s, pl.ds(0, GE)], wgf.at[pl.ds(0, GE)])

    plsc.subcore_barrier()

    for k in range(2):
        pltpu.async_copy(xw_hbm.at[srcg.at[0, k]], gbuf.at[k], gsems[k])

    def _chunk(j, k):
        ghj = (j // G) % 2
        slj = j % G
        p = k % 3
        pn = (k + 2) % 3
        p2 = k % 2

        jn = j + 2

        @pl.when(jn < NCHUNK)
        def _prefetch():
            pltpu.async_copy(xw_hbm.at[srcg.at[(jn // G) % 2, jn % G]],
                             gbuf.at[pn], gsems[pn])

        pltpu.make_async_copy(xw_hbm.at[srcg.at[0, 0]], gbuf.at[p],
                              gsems[p]).wait()

        @pl.when(j >= 2)
        def _wait_scatter():
            pltpu.make_async_copy(sbuf.at[p2], acc.at[dstg.at[0, 0]],
                                  ssems[p2]).wait()

        woff = GE * ghj + CH * slj

        @pl.loop(0, CH // 16)
        def _scale(g16):
            wv = wgf[pl.ds(woff + 16 * g16, 16)]

            @pl.loop(0, 16)
            def _row(r_):
                idxv = jnp.full((16,), r_, jnp.int32)
                ws = lax.gather(
                    wv, idxv[:, None],
                    lax.GatherDimensionNumbers(
                        offset_dims=(), collapsed_slice_dims=(0,),
                        start_index_map=(0,)),
                    slice_sizes=(1,),
                    mode=lax.GatherScatterMode.PROMISE_IN_BOUNDS)
                row = 16 * g16 + r_
                for v in range(8):
                    sl_ = pl.ds(16 * v, 16)
                    sbuf[p2, row, sl_] = gbuf[p, row, sl_] * ws

        pltpu.async_copy(sbuf.at[p2], acc.at[dstg.at[ghj, slj]],
                         ssems[p2], add=True)

    @pl.loop(0, NCHUNK, step=6)
    def _chunks(j0):
        gn = j0 // G + 1
        oth = gn % 2
        has_next = gn < NG

        @pl.when(has_next)
        def _stage_src():
            pltpu.sync_copy(srcb_hbm.at[c, s, gn], srcg.at[oth])
            pltpu.sync_copy(w_hbm.at[s, pl.ds(GE * gn, GE)],
                            wgf.at[pl.ds(GE * oth, GE)])

        _chunk(j0, 0)
        _chunk(j0 + 1, 1)

        @pl.when(has_next)
        def _stage_dst():
            pltpu.sync_copy(dst_hbm.at[s, gn], dstg.at[oth])

        _chunk(j0 + 2, 2)
        _chunk(j0 + 3, 3)
        _chunk(j0 + 4, 4)
        _chunk(j0 + 5, 5)

    for k in range(2):
        pltpu.make_async_copy(sbuf.at[k], acc.at[dstg.at[0, 0]],
                              ssems[k]).wait()

    plsc.subcore_barrier()

    col0 = pl.multiple_of(c * H, H)
    pltpu.sync_copy(acc.at[pl.ds(row0, ROWS_PT)],
                    out_hbm.at[pl.ds(row0, ROWS_PT), pl.ds(col0, H)])

    @pl.when(s == NT - 1)
    def _copy_tail():
        pltpu.sync_copy(acc.at[pl.ds(TAIL0, TAIL)],
                        out_hbm.at[pl.ds(TAIL0, TAIL), pl.ds(col0, H)])


def kernel(x, edge_index, edge_weight, W, b):
    src = edge_index[1].astype(jnp.int32)
    dst = edge_index[0].astype(jnp.int32)
    w = edge_weight.astype(jnp.float32)

    pad = EPAD - E
    pad_idx = (jnp.arange(pad, dtype=jnp.int32) * 61) % N
    src_p = jnp.concatenate([src, pad_idx]).reshape(NT, NG, G, CH)
    dst_p = jnp.concatenate([dst, pad_idx]).reshape(NT, NG, G, CH)
    w_p = jnp.concatenate([w, jnp.zeros((pad,), jnp.float32)])
    w_p = w_p.reshape(NT, EPT)
    srcb = jnp.stack([src_p, src_p + N])
    brep = jnp.broadcast_to(b.reshape(NC, 1, H), (NC, BR, H))

    xw = _matmul_halves(x, W)
    return _sc_aggregate(xw, srcb, dst_p, w_p, brep)


# speedup vs baseline: 2.6175x; 2.6175x over previous
import functools

import jax
import jax.numpy as jnp
from jax import lax
from jax.experimental import pallas as pl
from jax.experimental.pallas import tpu as pltpu
from jax.experimental.pallas import tpu_sc as plsc

N = 10000
E = 160000
D_IN = 256
D_OUT = 256
H = 128
NT = 16
NC = 2
CH = 64
NCHUNK = 160
NS = 5
G = 5
NG = NCHUNK // G
GE = G * CH
GEP = 384
EPT = CH * NCHUNK
EPAD = NT * EPT
ROWS_PT = 624
BR = 640
TAIL0 = NT * ROWS_PT
TAIL = N - TAIL0
MM_BLK = 1000
MM_NB = N // MM_BLK


def _mm_body(x_ref, w_ref, o_ref):
    o_ref[...] = jnp.dot(x_ref[...], w_ref[...],
                         preferred_element_type=jnp.float32)


def _matmul_halves(x, W):
    return pl.pallas_call(
        _mm_body,
        grid=(MM_NB, NC),
        in_specs=[
            pl.BlockSpec((MM_BLK, D_IN), lambda i, h: (i, 0)),
            pl.BlockSpec((D_IN, H), lambda i, h: (0, h)),
        ],
        out_specs=pl.BlockSpec((MM_BLK, H), lambda i, h: (h * MM_NB + i, 0)),
        out_shape=jax.ShapeDtypeStruct((NC * N, H), jnp.float32),
    )(x, W)


_MESH = plsc.VectorSubcoreMesh(core_axis_name="c", subcore_axis_name="s")


@functools.partial(
    pl.kernel,
    out_type=jax.ShapeDtypeStruct((N, D_OUT), jnp.float32),
    mesh=_MESH,
    compiler_params=pltpu.CompilerParams(internal_scratch_in_bytes=4096),
    scratch_types=[
        pltpu.VMEM((NS, CH, H), jnp.float32),
        pltpu.VMEM((2, G, CH), jnp.int32),
        pltpu.VMEM((2, G, CH), jnp.int32),
        pltpu.VMEM((2 * GEP,), jnp.float32),
        pltpu.VMEM_SHARED((N, H), jnp.float32),
        pltpu.SemaphoreType.DMA,
        pltpu.SemaphoreType.DMA,
        pltpu.SemaphoreType.DMA,
        pltpu.SemaphoreType.DMA,
        pltpu.SemaphoreType.DMA,
        pltpu.SemaphoreType.DMA,
        pltpu.SemaphoreType.DMA,
        pltpu.SemaphoreType.DMA,
        pltpu.SemaphoreType.DMA,
        pltpu.SemaphoreType.DMA,
    ],
)
def _sc_aggregate(xw_hbm, srcb_hbm, dst_hbm, w_hbm, brep_hbm, out_hbm,
                  gbuf, srcg, dstg, wgf, acc,
                  gsem0, gsem1, gsem2, gsem3, gsem4,
                  ssem0, ssem1, ssem2, ssem3, ssem4):
    c = lax.axis_index("c")
    s = lax.axis_index("s")
    gsems = (gsem0, gsem1, gsem2, gsem3, gsem4)
    ssems = (ssem0, ssem1, ssem2, ssem3, ssem4)

    row0 = s * ROWS_PT
    pltpu.sync_copy(brep_hbm.at[c, pl.ds(0, ROWS_PT)],
                    acc.at[pl.ds(row0, ROWS_PT)])

    @pl.when(s == NT - 1)
    def _init_tail():
        pltpu.sync_copy(brep_hbm.at[c, pl.ds(ROWS_PT, TAIL)],
                        acc.at[pl.ds(TAIL0, TAIL)])

    pltpu.sync_copy(srcb_hbm.at[c, s, 0], srcg.at[0])
    pltpu.sync_copy(dst_hbm.at[s, 0], dstg.at[0])
    pltpu.sync_copy(w_hbm.at[s, 0], wgf.at[pl.ds(0, GEP)])

    plsc.subcore_barrier()

    for k in range(3):
        pltpu.async_copy(xw_hbm.at[srcg.at[0, k]], gbuf.at[k], gsems[k])

    def _chunk(j, p):
        ghj = (j // G) % 2
        slj = j % G
        pn = (p + 3) % NS

        pltpu.make_async_copy(xw_hbm.at[srcg.at[0, 0]], gbuf.at[p],
                              gsems[p]).wait()

        woff = GEP * ghj + CH * slj

        @pl.loop(0, CH // 16)
        def _scale(g16):
            wv = wgf[pl.ds(woff + 16 * g16, 16)]
            for r_ in range(16):
                ws = wv[r_]
                row = 16 * g16 + r_
                for v in range(8):
                    sl_ = pl.ds(16 * v, 16)
                    gbuf[p, row, sl_] = gbuf[p, row, sl_] * ws

        pltpu.async_copy(gbuf.at[p], acc.at[dstg.at[ghj, slj]],
                         ssems[p], add=True)

        jn = j + 3

        @pl.when(jnp.logical_and(j >= 2, jn < NCHUNK))
        def _wait_scatter():
            pltpu.make_async_copy(gbuf.at[pn], acc.at[dstg.at[0, 0]],
                                  ssems[pn]).wait()

        @pl.when(jn < NCHUNK)
        def _prefetch():
            pltpu.async_copy(xw_hbm.at[srcg.at[(jn // G) % 2, jn % G]],
                             gbuf.at[pn], gsems[pn])

    @pl.loop(0, NCHUNK, step=G)
    def _chunks(j0):
        gn = j0 // G + 1
        oth = gn % 2
        has_next = gn < NG

        @pl.when(has_next)
        def _stage_src():
            pltpu.sync_copy(srcb_hbm.at[c, s, gn], srcg.at[oth])
            pltpu.sync_copy(w_hbm.at[s, gn], wgf.at[pl.ds(GEP * oth, GEP)])

        _chunk(j0, 0)
        _chunk(j0 + 1, 1)

        @pl.when(has_next)
        def _stage_dst():
            pltpu.sync_copy(dst_hbm.at[s, gn], dstg.at[oth])

        _chunk(j0 + 2, 2)
        _chunk(j0 + 3, 3)
        _chunk(j0 + 4, 4)

    for k in range(NS):
        pltpu.make_async_copy(gbuf.at[k], acc.at[dstg.at[0, 0]],
                              ssems[k]).wait()

    plsc.subcore_barrier()

    col0 = pl.multiple_of(c * H, H)
    pltpu.sync_copy(acc.at[pl.ds(row0, ROWS_PT)],
                    out_hbm.at[pl.ds(row0, ROWS_PT), pl.ds(col0, H)])

    @pl.when(s == NT - 1)
    def _copy_tail():
        pltpu.sync_copy(acc.at[pl.ds(TAIL0, TAIL)],
                        out_hbm.at[pl.ds(TAIL0, TAIL), pl.ds(col0, H)])


def kernel(x, edge_index, edge_weight, W, b):
    src = edge_index[1].astype(jnp.int32)
    dst = edge_index[0].astype(jnp.int32)
    w = edge_weight.astype(jnp.float32)

    pad = EPAD - E
    pad_idx = (jnp.arange(pad, dtype=jnp.int32) * 61) % N
    src_p = jnp.concatenate([src, pad_idx]).reshape(NT, NG, G, CH)
    dst_p = jnp.concatenate([dst, pad_idx]).reshape(NT, NG, G, CH)
    w_p = jnp.concatenate([w, jnp.zeros((pad,), jnp.float32)])
    w_p = jnp.pad(w_p.reshape(NT, NG, GE), ((0, 0), (0, 0), (0, GEP - GE)))
    srcb = jnp.stack([src_p, src_p + N])
    brep = jnp.broadcast_to(b.reshape(NC, 1, H), (NC, BR, H))

    xw = _matmul_halves(x, W)
    return _sc_aggregate(xw, srcb, dst_p, w_p, brep)


# speedup vs baseline: 2.6932x; 1.0289x over previous
import functools

import jax
import jax.numpy as jnp
from jax import lax
from jax.experimental import pallas as pl
from jax.experimental.pallas import tpu as pltpu
from jax.experimental.pallas import tpu_sc as plsc

N = 10000
E = 160000
D_IN = 256
D_OUT = 256
H = 128
NT = 16
NC = 2
CH = 64
NCHUNK = 160
G = 8
NG = NCHUNK // G
GE = G * CH
EPT = CH * NCHUNK
EPAD = NT * EPT
ROWS_PT = 624
BR = 640
TAIL0 = NT * ROWS_PT
TAIL = N - TAIL0
MM_BLK = 1000
MM_NB = N // MM_BLK


def _mm_body(x_ref, w_ref, o_ref):
    o_ref[...] = jnp.dot(x_ref[...], w_ref[...],
                         preferred_element_type=jnp.float32)


def _matmul_halves(x, W):
    return pl.pallas_call(
        _mm_body,
        grid=(MM_NB, NC),
        in_specs=[
            pl.BlockSpec((MM_BLK, D_IN), lambda i, h: (i, 0)),
            pl.BlockSpec((D_IN, H), lambda i, h: (0, h)),
        ],
        out_specs=pl.BlockSpec((MM_BLK, H), lambda i, h: (h * MM_NB + i, 0)),
        out_shape=jax.ShapeDtypeStruct((NC * N, H), jnp.float32),
    )(x, W)


_MESH = plsc.VectorSubcoreMesh(core_axis_name="c", subcore_axis_name="s")


@functools.partial(
    pl.kernel,
    out_type=jax.ShapeDtypeStruct((N, D_OUT), jnp.float32),
    mesh=_MESH,
    scratch_types=[
        pltpu.VMEM((4, CH, H), jnp.float32),
        pltpu.VMEM((2, G, CH), jnp.int32),
        pltpu.VMEM((2, G, CH), jnp.int32),
        pltpu.VMEM((2 * GE,), jnp.float32),
        pltpu.VMEM_SHARED((N, H), jnp.float32),
        pltpu.SemaphoreType.DMA,
        pltpu.SemaphoreType.DMA,
        pltpu.SemaphoreType.DMA,
        pltpu.SemaphoreType.DMA,
        pltpu.SemaphoreType.DMA,
        pltpu.SemaphoreType.DMA,
        pltpu.SemaphoreType.DMA,
        pltpu.SemaphoreType.DMA,
    ],
)
def _sc_aggregate(xw_hbm, srcb_hbm, dst_hbm, w_hbm, brep_hbm, out_hbm,
                  gbuf, srcg, dstg, wgf, acc,
                  gsem0, gsem1, gsem2, gsem3, ssem0, ssem1, ssem2, ssem3):
    c = lax.axis_index("c")
    s = lax.axis_index("s")
    gsems = (gsem0, gsem1, gsem2, gsem3)
    ssems = (ssem0, ssem1, ssem2, ssem3)

    row0 = s * ROWS_PT
    pltpu.sync_copy(brep_hbm.at[c, pl.ds(0, ROWS_PT)],
                    acc.at[pl.ds(row0, ROWS_PT)])

    @pl.when(s == NT - 1)
    def _init_tail():
        pltpu.sync_copy(brep_hbm.at[c, pl.ds(ROWS_PT, TAIL)],
                        acc.at[pl.ds(TAIL0, TAIL)])

    pltpu.sync_copy(srcb_hbm.at[c, s, pl.ds(0, G)], srcg.at[0])
    pltpu.sync_copy(dst_hbm.at[s, pl.ds(0, G)], dstg.at[0])
    pltpu.sync_copy(w_hbm.at[s, pl.ds(0, GE)], wgf.at[pl.ds(0, GE)])

    plsc.subcore_barrier()

    for k in range(3):
        pltpu.async_copy(xw_hbm.at[srcg.at[0, k]], gbuf.at[k], gsems[k])

    def _chunk(j, p):
        ghj = (j // G) % 2
        slj = j % G
        pn = (p + 3) % 4

        pltpu.make_async_copy(xw_hbm.at[srcg.at[0, 0]], gbuf.at[p],
                              gsems[p]).wait()

        woff = GE * ghj + CH * slj

        @pl.loop(0, CH // 16)
        def _scale(g16):
            wv = wgf[pl.ds(woff + 16 * g16, 16)]
            for r_ in range(16):
                ws = wv[r_]
                row = 16 * g16 + r_
                for v in range(8):
                    sl_ = pl.ds(16 * v, 16)
                    gbuf[p, row, sl_] = gbuf[p, row, sl_] * ws

        pltpu.async_copy(gbuf.at[p], acc.at[dstg.at[ghj, slj]],
                         ssems[p], add=True)

        jn = j + 3

        @pl.when(jnp.logical_and(j >= 1, jn < NCHUNK))
        def _wait_scatter():
            pltpu.make_async_copy(gbuf.at[pn], acc.at[dstg.at[0, 0]],
                                  ssems[pn]).wait()

        @pl.when(jn < NCHUNK)
        def _prefetch():
            pltpu.async_copy(xw_hbm.at[srcg.at[(jn // G) % 2, jn % G]],
                             gbuf.at[pn], gsems[pn])

    @pl.loop(0, NCHUNK, step=4)
    def _chunks(j0):
        @pl.when(jnp.logical_and(j0 % G == 4, j0 // G + 1 < NG))
        def _stage():
            gn = j0 // G + 1
            oth = gn % 2
            pltpu.sync_copy(srcb_hbm.at[c, s, pl.ds(G * gn, G)],
                            srcg.at[oth])
            pltpu.sync_copy(dst_hbm.at[s, pl.ds(G * gn, G)], dstg.at[oth])
            pltpu.sync_copy(w_hbm.at[s, pl.ds(GE * gn, GE)],
                            wgf.at[pl.ds(GE * oth, GE)])

        for k in range(4):
            _chunk(j0 + k, k)

    for k in range(4):
        pltpu.make_async_copy(gbuf.at[k], acc.at[dstg.at[0, 0]],
                              ssems[k]).wait()

    plsc.subcore_barrier()

    col0 = pl.multiple_of(c * H, H)
    pltpu.sync_copy(acc.at[pl.ds(row0, ROWS_PT)],
                    out_hbm.at[pl.ds(row0, ROWS_PT), pl.ds(col0, H)])

    @pl.when(s == NT - 1)
    def _copy_tail():
        pltpu.sync_copy(acc.at[pl.ds(TAIL0, TAIL)],
                        out_hbm.at[pl.ds(TAIL0, TAIL), pl.ds(col0, H)])


def kernel(x, edge_index, edge_weight, W, b):
    src = edge_index[1].astype(jnp.int32)
    dst = edge_index[0].astype(jnp.int32)
    w = edge_weight.astype(jnp.float32)

    pad = EPAD - E
    pad_idx = (jnp.arange(pad, dtype=jnp.int32) * 61) % N
    src_p = jnp.concatenate([src, pad_idx]).reshape(NT, NCHUNK, CH)
    dst_p = jnp.concatenate([dst, pad_idx]).reshape(NT, NCHUNK, CH)
    w_p = jnp.concatenate([w, jnp.zeros((pad,), jnp.float32)])
    w_p = w_p.reshape(NT, EPT)
    srcb = jnp.stack([src_p, src_p + N])
    brep = jnp.broadcast_to(b.reshape(NC, 1, H), (NC, BR, H))

    xw = _matmul_halves(x, W)
    return _sc_aggregate(xw, srcb, dst_p, w_p, brep)


# speedup vs baseline: 2.7132x; 1.0074x over previous
import functools

import jax
import jax.numpy as jnp
from jax import lax
from jax.experimental import pallas as pl
from jax.experimental.pallas import tpu as pltpu
from jax.experimental.pallas import tpu_sc as plsc

N = 10000
E = 160000
D_IN = 256
D_OUT = 256
H = 128
NT = 16
NC = 2
CH = 64
NCHUNK = 160
G = 8
NG = NCHUNK // G
GE = G * CH
EPT = CH * NCHUNK
EPAD = NT * EPT
ROWS_PT = 624
OR = 16
NPASS = ROWS_PT // OR
TAIL0 = NT * ROWS_PT
TAIL = N - TAIL0
MM_BLK = 1000
MM_NB = N // MM_BLK


def _mm_body(x_ref, w_ref, o_ref):
    o_ref[...] = jnp.dot(x_ref[...], w_ref[...],
                         preferred_element_type=jnp.float32)


def _matmul_halves(x, W):
    return pl.pallas_call(
        _mm_body,
        grid=(MM_NB, NC),
        in_specs=[
            pl.BlockSpec((MM_BLK, D_IN), lambda i, h: (i, 0)),
            pl.BlockSpec((D_IN, H), lambda i, h: (0, h)),
        ],
        out_specs=pl.BlockSpec((MM_BLK, H), lambda i, h: (h * MM_NB + i, 0)),
        out_shape=jax.ShapeDtypeStruct((NC * N, H), jnp.float32),
    )(x, W)


_MESH = plsc.VectorSubcoreMesh(core_axis_name="c", subcore_axis_name="s")


@functools.partial(
    pl.kernel,
    out_type=jax.ShapeDtypeStruct((N, D_OUT), jnp.float32),
    mesh=_MESH,
    scratch_types=[
        pltpu.VMEM((4, CH, H), jnp.float32),
        pltpu.VMEM((2, G, CH), jnp.int32),
        pltpu.VMEM((2, G, CH), jnp.int32),
        pltpu.VMEM((2 * GE,), jnp.float32),
        pltpu.VMEM((OR, H), jnp.float32),
        pltpu.VMEM((NC, H), jnp.float32),
        pltpu.VMEM_SHARED((N, H), jnp.float32),
        pltpu.SemaphoreType.DMA,
        pltpu.SemaphoreType.DMA,
        pltpu.SemaphoreType.DMA,
        pltpu.SemaphoreType.DMA,
        pltpu.SemaphoreType.DMA,
        pltpu.SemaphoreType.DMA,
        pltpu.SemaphoreType.DMA,
        pltpu.SemaphoreType.DMA,
    ],
)
def _sc_aggregate(xw_hbm, srcb_hbm, dst_hbm, w_hbm, b_hbm, out_hbm,
                  gbuf, srcg, dstg, wgf, obuf, bbuf, acc,
                  gsem0, gsem1, gsem2, gsem3, ssem0, ssem1, ssem2, ssem3):
    c = lax.axis_index("c")
    s = lax.axis_index("s")
    gsems = (gsem0, gsem1, gsem2, gsem3)
    ssems = (ssem0, ssem1, ssem2, ssem3)

    pltpu.sync_copy(b_hbm, bbuf)
    bvecs = [bbuf[c, pl.ds(16 * k, 16)] for k in range(8)]

    @pl.loop(0, OR)
    def _fill(r):
        for k in range(8):
            obuf[r, pl.ds(16 * k, 16)] = bvecs[k]

    row0 = s * ROWS_PT

    @pl.loop(0, NPASS)
    def _init(p):
        pltpu.sync_copy(obuf, acc.at[pl.ds(row0 + p * OR, OR)])

    @pl.when(s == NT - 1)
    def _init_tail():
        pltpu.sync_copy(obuf.at[pl.ds(0, TAIL)], acc.at[pl.ds(TAIL0, TAIL)])

    pltpu.sync_copy(srcb_hbm.at[c, s, pl.ds(0, G)], srcg.at[0])
    pltpu.sync_copy(dst_hbm.at[s, pl.ds(0, G)], dstg.at[0])
    pltpu.sync_copy(w_hbm.at[s, pl.ds(0, GE)], wgf.at[pl.ds(0, GE)])

    plsc.subcore_barrier()

    for k in range(3):
        pltpu.async_copy(xw_hbm.at[srcg.at[0, k]], gbuf.at[k], gsems[k])

    def _chunk(j, p):
        ghj = (j // G) % 2
        slj = j % G
        pn = (p + 3) % 4

        pltpu.make_async_copy(xw_hbm.at[srcg.at[0, 0]], gbuf.at[p],
                              gsems[p]).wait()

        woff = GE * ghj + CH * slj

        @pl.loop(0, CH // 16)
        def _scale(g16):
            wv = wgf[pl.ds(woff + 16 * g16, 16)]
            for r_ in range(16):
                ws = wv[r_]
                row = 16 * g16 + r_
                for v in range(8):
                    sl_ = pl.ds(16 * v, 16)
                    gbuf[p, row, sl_] = gbuf[p, row, sl_] * ws

        pltpu.async_copy(gbuf.at[p], acc.at[dstg.at[ghj, slj]],
                         ssems[p], add=True)

        jn = j + 3

        @pl.when(jnp.logical_and(j >= 1, jn < NCHUNK))
        def _wait_scatter():
            pltpu.make_async_copy(gbuf.at[pn], acc.at[dstg.at[0, 0]],
                                  ssems[pn]).wait()

        @pl.when(jn < NCHUNK)
        def _prefetch():
            pltpu.async_copy(xw_hbm.at[srcg.at[(jn // G) % 2, jn % G]],
                             gbuf.at[pn], gsems[pn])

    @pl.loop(0, NCHUNK, step=4)
    def _chunks(j0):
        @pl.when(jnp.logical_and(j0 % G == 4, j0 // G + 1 < NG))
        def _stage():
            gn = j0 // G + 1
            oth = gn % 2
            pltpu.sync_copy(srcb_hbm.at[c, s, pl.ds(G * gn, G)],
                            srcg.at[oth])
            pltpu.sync_copy(dst_hbm.at[s, pl.ds(G * gn, G)], dstg.at[oth])
            pltpu.sync_copy(w_hbm.at[s, pl.ds(GE * gn, GE)],
                            wgf.at[pl.ds(GE * oth, GE)])

        for k in range(4):
            _chunk(j0 + k, k)

    for k in range(4):
        pltpu.make_async_copy(gbuf.at[k], acc.at[dstg.at[0, 0]],
                              ssems[k]).wait()

    plsc.subcore_barrier()

    col0 = pl.multiple_of(c * H, H)
    pltpu.sync_copy(acc.at[pl.ds(row0, ROWS_PT)],
                    out_hbm.at[pl.ds(row0, ROWS_PT), pl.ds(col0, H)])

    @pl.when(s == NT - 1)
    def _copy_tail():
        pltpu.sync_copy(acc.at[pl.ds(TAIL0, TAIL)],
                        out_hbm.at[pl.ds(TAIL0, TAIL), pl.ds(col0, H)])


def kernel(x, edge_index, edge_weight, W, b):
    src = edge_index[1].astype(jnp.int32)
    dst = edge_index[0].astype(jnp.int32)
    w = edge_weight.astype(jnp.float32)

    pad = EPAD - E
    pad_idx = (jnp.arange(pad, dtype=jnp.int32) * 61) % N
    src_p = jnp.concatenate([src, pad_idx]).reshape(NT, NCHUNK, CH)
    dst_p = jnp.concatenate([dst, pad_idx]).reshape(NT, NCHUNK, CH)
    w_p = jnp.concatenate([w, jnp.zeros((pad,), jnp.float32)])
    w_p = w_p.reshape(NT, EPT)
    srcb = jnp.stack([src_p, src_p + N])
    b2 = b.reshape(NC, H)

    xw = _matmul_halves(x, W)
    return _sc_aggregate(xw, srcb, dst_p, w_p, b2)
